# baseline (speedup 1.0000x reference)
import jax
import jax.numpy as jnp
from jax.experimental import pallas as pl
from jax.experimental.pallas import tpu as pltpu

N = 100
DIM = 128
HID = 60
ACT = 100
E = 1600

_NIN = 15


def _body(*refs):
    hbm = refs[:_NIN]
    lo_ref, vo_ref = refs[_NIN], refs[_NIN + 1]
    vmem = refs[_NIN + 2:2 * _NIN + 2]
    flat_a_ref, flat_c_ref, sems = refs[2 * _NIN + 2:]

    copies = [pltpu.make_async_copy(h, v, sems.at[i])
              for i, (h, v) in enumerate(zip(hbm, vmem))]
    iwfa = 11
    copies[iwfa].start()
    for i in range(_NIN):
        if i != iwfa:
            copies[i].start()
    for i in range(_NIN):
        if i != iwfa:
            copies[i].wait()

    (edge_ref, x_ref, vnr_ref, wa_ref, ba_ref, wc_ref, bc_ref,
     wav_ref, bav_ref, wcv_ref, bcv_ref, wfa_ref, bfa_ref, wfv_ref,
     bfv_ref) = vmem

    src = edge_ref[0:1, :]
    dst = edge_ref[1:2, :]
    ids = jax.lax.broadcasted_iota(jnp.int32, (N, E), 0)
    odst = (ids == dst).astype(jnp.float32)
    osrc = (ids == src).astype(jnp.float32)
    a = jax.lax.dot_general(odst, osrc, (((1,), (1,)), ((), ())),
                            preferred_element_type=jnp.float32)
    deg = jnp.sum(a, axis=1, keepdims=True)
    dis = jnp.where(deg > 0, jax.lax.rsqrt(jnp.maximum(deg, 1e-12)), 0.0)
    x = x_ref[...]
    hp = jax.lax.Precision.HIGHEST
    tx1 = -dis * jax.lax.dot(a, dis * x, precision=hp)
    tx2 = -2.0 * dis * jax.lax.dot(a, dis * tx1, precision=hp) - x
    vnr = vnr_ref[...]

    def branch(w3, b, wv, bv):
        g = jnp.tanh(jax.lax.dot(x, w3[0]) + jax.lax.dot(tx1, w3[1]) +
                     jax.lax.dot(tx2, w3[2]) + b.reshape(1, HID))
        vvec = (vnr[0, 0] * wv[0] + vnr[0, 1] * wv[1] + vnr[0, 2] * wv[2]
                + jnp.sum(bv, axis=0, keepdims=True))
        return g + vvec

    fa = branch(wa_ref[...], ba_ref[...], wav_ref[...], bav_ref[...])
    fc = branch(wc_ref[...], bc_ref[...], wcv_ref[...], bcv_ref[...])

    for n in range(N):
        flat_a_ref[:, n * HID:(n + 1) * HID] = fa[n:n + 1, :]
        flat_c_ref[:, n * HID:(n + 1) * HID] = fc[n:n + 1, :]

    copies[iwfa].wait()
    lo_ref[...] = (jax.lax.dot(flat_a_ref[...], wfa_ref[...])
                   + bfa_ref[...].reshape(1, ACT))
    vo_ref[...] = (jnp.sum(flat_c_ref[...] * wfv_ref[...])
                   + bfv_ref[0]).reshape(1, 1)


def kernel(substrate_features, substrate_edge_index, vnr_features,
           Wa, ba, Wc, bc, wav, bav, wcv, bcv, Wfa, bfa, Wfv, bfv):
    ins = (substrate_edge_index.astype(jnp.int32), substrate_features,
           vnr_features, Wa, ba, Wc, bc, wav, bav, wcv, bcv,
           Wfa, bfa, Wfv.reshape(1, N * HID), bfv)
    vmem_scratch = [pltpu.VMEM(i.shape, i.dtype) for i in ins]
    logits, values = pl.pallas_call(
        _body,
        out_shape=(jax.ShapeDtypeStruct((1, ACT), jnp.float32),
                   jax.ShapeDtypeStruct((1, 1), jnp.float32)),
        in_specs=[pl.BlockSpec(memory_space=pltpu.MemorySpace.HBM)] * _NIN,
        scratch_shapes=vmem_scratch + [
            pltpu.VMEM((1, N * HID), jnp.float32),
            pltpu.VMEM((1, N * HID), jnp.float32),
            pltpu.SemaphoreType.DMA((_NIN,)),
        ],
    )(*ins)
    return logits, values
